# baseline (speedup 1.0000x reference)
import jax
import jax.numpy as jnp
from jax import lax
from jax.experimental import pallas as pl
from jax.experimental.pallas import tpu as pltpu
from jax.experimental.pallas import tpu_sc as plsc

BATCH = 16384
GENDER_VOCAB = 3
AGE_VOCAB = 10
EMBED_DIM = 32
OUT_DIM = 2 * EMBED_DIM
NUM_COMBOS = GENDER_VOCAB * AGE_VOCAB

_INFO = plsc.get_sparse_core_info()
NUM_CORES = _INFO.num_cores
NUM_SUBCORES = _INFO.num_subcores
NUM_WORKERS = NUM_CORES * NUM_SUBCORES
BPW = BATCH // NUM_WORKERS
CHUNK = 128
NCHUNK = BPW // CHUNK
LANES = 16
GROUPS_PER_CHUNK = CHUNK // LANES


def _sc_lookup(gender_table, age_table, gender_ids, age_ids):
    mesh = plsc.VectorSubcoreMesh(core_axis_name="c", subcore_axis_name="s")

    def body(gt_hbm, at_hbm, g_hbm, a_hbm, out_hbm, gt_v, at_v, tab_v,
             gid_v, aid_v, cb_v, rows_v, sem_in, sem_out):
        wid = lax.axis_index("s") * NUM_CORES + lax.axis_index("c")
        base = wid * BPW
        gt_copy = pltpu.async_copy(gt_hbm, gt_v, sem_in)
        at_copy = pltpu.async_copy(at_hbm, at_v, sem_in)
        gi_copy = pltpu.async_copy(g_hbm.at[pl.ds(base, BPW)], gid_v, sem_in)
        ai_copy = pltpu.async_copy(a_hbm.at[pl.ds(base, BPW)], aid_v, sem_in)
        gt_copy.wait()
        at_copy.wait()
        for c in range(NUM_COMBOS):
            g, a = divmod(c, AGE_VOCAB)
            for k in range(EMBED_DIM // LANES):
                tab_v[pl.ds(c * OUT_DIM + k * LANES, LANES)] = (
                    gt_v[g, pl.ds(k * LANES, LANES)]
                )
                tab_v[pl.ds(c * OUT_DIM + EMBED_DIM + k * LANES, LANES)] = (
                    at_v[a, pl.ds(k * LANES, LANES)]
                )
        gi_copy.wait()
        ai_copy.wait()
        for k in range(BPW // LANES):
            sl = pl.ds(k * LANES, LANES)
            cb_v[sl] = gid_v[sl] * (AGE_VOCAB * OUT_DIM) + aid_v[sl] * OUT_DIM

        out_copies = []
        for ch in range(NCHUNK):

            def group_body(grp, _, ch=ch):
                row0 = ch * CHUNK + grp * LANES
                cb = cb_v[pl.ds(row0, LANES)]
                for l in range(LANES):
                    b = cb[l]
                    for k in range(OUT_DIM // LANES):
                        rows_v[row0 + l, pl.ds(k * LANES, LANES)] = (
                            tab_v[pl.ds(b + k * LANES, LANES)]
                        )
                return _

            lax.fori_loop(0, GROUPS_PER_CHUNK, group_body, 0, unroll=False)
            out_copies.append(
                pltpu.async_copy(
                    rows_v.at[pl.ds(ch * CHUNK, CHUNK)],
                    out_hbm.at[pl.ds(base + ch * CHUNK, CHUNK)],
                    sem_out,
                )
            )
        for c in out_copies:
            c.wait()

    return pl.kernel(
        body,
        out_type=jax.ShapeDtypeStruct((BATCH, OUT_DIM), jnp.float32),
        mesh=mesh,
        compiler_params=pltpu.CompilerParams(use_tc_tiling_on_sc=False),
        scratch_types=[
            pltpu.VMEM((GENDER_VOCAB, EMBED_DIM), jnp.float32),
            pltpu.VMEM((AGE_VOCAB, EMBED_DIM), jnp.float32),
            pltpu.VMEM((NUM_COMBOS * OUT_DIM,), jnp.float32),
            pltpu.VMEM((BPW,), jnp.int32),
            pltpu.VMEM((BPW,), jnp.int32),
            pltpu.VMEM((BPW,), jnp.int32),
            pltpu.VMEM((BPW, OUT_DIM), jnp.float32),
            pltpu.SemaphoreType.DMA,
            pltpu.SemaphoreType.DMA,
        ],
    )(gender_table, age_table, gender_ids, age_ids)


def kernel(gender_ids, age_ids, gender_table, age_table):
    return _sc_lookup(
        gender_table,
        age_table,
        gender_ids.astype(jnp.int32),
        age_ids.astype(jnp.int32),
    )


# speedup vs baseline: 1.1874x; 1.1874x over previous
import jax
import jax.numpy as jnp
from jax import lax
from jax.experimental import pallas as pl
from jax.experimental.pallas import tpu as pltpu
from jax.experimental.pallas import tpu_sc as plsc

BATCH = 16384
GENDER_VOCAB = 3
AGE_VOCAB = 10
EMBED_DIM = 32
OUT_DIM = 2 * EMBED_DIM
NUM_COMBOS = GENDER_VOCAB * AGE_VOCAB

_INFO = plsc.get_sparse_core_info()
NUM_CORES = _INFO.num_cores
NUM_SUBCORES = _INFO.num_subcores
NUM_WORKERS = NUM_CORES * NUM_SUBCORES
BPW = BATCH // NUM_WORKERS
CHUNK = 128
NCHUNK = BPW // CHUNK
LANES = 16
GROUPS_PER_CHUNK = CHUNK // LANES


def _sc_lookup(gender_table, age_table, gender_ids, age_ids):
    mesh = plsc.VectorSubcoreMesh(core_axis_name="c", subcore_axis_name="s")

    def body(gt_hbm, at_hbm, g_hbm, a_hbm, out_hbm, gt_v, at_v, tab_v,
             gid_v, aid_v, cb_v, rows_v, sem_in, sem_out):
        wid = lax.axis_index("s") * NUM_CORES + lax.axis_index("c")
        base = wid * BPW
        gt_copy = pltpu.async_copy(gt_hbm, gt_v, sem_in)
        at_copy = pltpu.async_copy(at_hbm, at_v, sem_in)
        gi_copy = pltpu.async_copy(g_hbm.at[pl.ds(base, BPW)], gid_v, sem_in)
        ai_copy = pltpu.async_copy(a_hbm.at[pl.ds(base, BPW)], aid_v, sem_in)
        gt_copy.wait()
        at_copy.wait()
        for c in range(NUM_COMBOS):
            g, a = divmod(c, AGE_VOCAB)
            for k in range(EMBED_DIM // LANES):
                tab_v[pl.ds(c * OUT_DIM + k * LANES, LANES)] = (
                    gt_v[g, pl.ds(k * LANES, LANES)]
                )
                tab_v[pl.ds(c * OUT_DIM + EMBED_DIM + k * LANES, LANES)] = (
                    at_v[a, pl.ds(k * LANES, LANES)]
                )
        gi_copy.wait()
        ai_copy.wait()
        for k in range(BPW // LANES):
            sl = pl.ds(k * LANES, LANES)
            cb_v[sl] = gid_v[sl] * (AGE_VOCAB * OUT_DIM) + aid_v[sl] * OUT_DIM

        out_copies = []
        for ch in range(NCHUNK):

            def group_body(grp, _, ch=ch):
                row0 = ch * CHUNK + grp * LANES
                cb = cb_v[pl.ds(row0, LANES)]
                for l in range(LANES):
                    b = cb[l]
                    for k in range(OUT_DIM // LANES):
                        rows_v[row0 + l, pl.ds(k * LANES, LANES)] = (
                            tab_v[pl.ds(b + k * LANES, LANES)]
                        )
                return _

            lax.fori_loop(0, GROUPS_PER_CHUNK, group_body, 0, unroll=False)
            out_copies.append(
                pltpu.async_copy(
                    rows_v.at[pl.ds(ch * CHUNK, CHUNK)],
                    out_hbm.at[pl.ds(base + ch * CHUNK, CHUNK)],
                    sem_out,
                )
            )
        for c in out_copies:
            c.wait()

    return pl.kernel(
        body,
        out_type=jax.ShapeDtypeStruct((BATCH, OUT_DIM), jnp.float32),
        mesh=mesh,
        compiler_params=pltpu.CompilerParams(use_tc_tiling_on_sc=True),
        scratch_types=[
            pltpu.VMEM((GENDER_VOCAB, EMBED_DIM), jnp.float32),
            pltpu.VMEM((AGE_VOCAB, EMBED_DIM), jnp.float32),
            pltpu.VMEM((NUM_COMBOS * OUT_DIM,), jnp.float32),
            pltpu.VMEM((BPW,), jnp.int32),
            pltpu.VMEM((BPW,), jnp.int32),
            pltpu.VMEM((BPW,), jnp.int32),
            pltpu.VMEM((BPW, OUT_DIM), jnp.float32),
            pltpu.SemaphoreType.DMA,
            pltpu.SemaphoreType.DMA,
        ],
    )(gender_table, age_table, gender_ids, age_ids)


def kernel(gender_ids, age_ids, gender_table, age_table):
    return _sc_lookup(
        gender_table,
        age_table,
        gender_ids.astype(jnp.int32),
        age_ids.astype(jnp.int32),
    )
